# baseline (speedup 1.0000x reference)
import jax
import jax.numpy as jnp
from jax.experimental import pallas as pl
from jax.experimental.pallas import tpu as pltpu

BATCH = 1024
CAST = 256
HID = 128
INP = 128
BLK = 32


def _fused_body(x_ref, st_ref, wiT_ref, whT_ref, bi_ref, bh_ref, aid_ref,
                stop_ref, sel_ref, out_ref):
    g = pl.program_id(0)
    base = g * BLK
    x = x_ref[...]

    rows = []
    for r_i in range(BLK):
        a = jnp.clip(aid_ref[base + r_i], 0, CAST - 1)
        rows.append(st_ref[r_i, pl.ds(a, 1), :])
    h = jnp.concatenate(rows, axis=0)

    dn = (((1,), (1,)), ((), ()))
    gi = jax.lax.dot_general(x, wiT_ref[...], dn,
                             preferred_element_type=jnp.float32) + bi_ref[...]
    gh = jax.lax.dot_general(h, whT_ref[...], dn,
                             preferred_element_type=jnp.float32) + bh_ref[...]
    i_r, i_z, i_n = gi[:, :HID], gi[:, HID:2 * HID], gi[:, 2 * HID:]
    h_r, h_z, h_n = gh[:, :HID], gh[:, HID:2 * HID], gh[:, 2 * HID:]
    r = jax.nn.sigmoid(i_r + h_r)
    z = jax.nn.sigmoid(i_z + h_z)
    n = jnp.tanh(i_n + r * h_n)
    new_h = (1.0 - z) * n + z * h
    sel_ref[...] = new_h

    out_ref[...] = st_ref[...]
    for r_i in range(BLK):
        a = jnp.clip(aid_ref[base + r_i], 0, CAST - 1)
        out_ref[r_i, pl.ds(a, 1), :] = new_h[r_i:r_i + 1, :]
    n_stop = stop_ref.shape[0]
    for r_i in range(BLK):
        rid = base + r_i
        cond = stop_ref[0] == rid
        for j in range(1, n_stop):
            cond = jnp.logical_or(cond, stop_ref[j] == rid)

        def _zero(r_i=r_i):
            out_ref[r_i] = jnp.zeros((CAST, HID), jnp.float32)
        pl.when(cond)(_zero)


def kernel(x, state, W_ih, W_hh, b_ih, b_hh, batch_idxs, actor_ids,
           story_stop_idxs):
    del batch_idxs
    aid = actor_ids.astype(jnp.int32)
    stops = story_stop_idxs.astype(jnp.int32)
    bi = b_ih.reshape(1, 3 * HID)
    bh = b_hh.reshape(1, 3 * HID)

    grid = BATCH // BLK
    new_selected, new_state = pl.pallas_call(
        _fused_body,
        grid=(grid,),
        in_specs=[
            pl.BlockSpec((BLK, INP), lambda g: (g, 0)),
            pl.BlockSpec((BLK, CAST, HID), lambda g: (g, 0, 0)),
            pl.BlockSpec((3 * HID, INP), lambda g: (0, 0)),
            pl.BlockSpec((3 * HID, HID), lambda g: (0, 0)),
            pl.BlockSpec((1, 3 * HID), lambda g: (0, 0)),
            pl.BlockSpec((1, 3 * HID), lambda g: (0, 0)),
            pl.BlockSpec(memory_space=pltpu.SMEM),
            pl.BlockSpec(memory_space=pltpu.SMEM),
        ],
        out_specs=[
            pl.BlockSpec((BLK, HID), lambda g: (g, 0)),
            pl.BlockSpec((BLK, CAST, HID), lambda g: (g, 0, 0)),
        ],
        out_shape=[
            jax.ShapeDtypeStruct((BATCH, HID), jnp.float32),
            jax.ShapeDtypeStruct((BATCH, CAST, HID), jnp.float32),
        ],
    )(x, state, W_ih, W_hh, bi, bh, aid, stops)
    return new_selected, new_state


# speedup vs baseline: 1.0255x; 1.0255x over previous
import jax
import jax.numpy as jnp
from jax.experimental import pallas as pl
from jax.experimental.pallas import tpu as pltpu

BATCH = 1024
CAST = 256
HID = 128
INP = 128
BLK = 64


def _fused_body(x_ref, st_ref, wiT_ref, whT_ref, bi_ref, bh_ref, aid_ref,
                stop_ref, sel_ref, out_ref):
    g = pl.program_id(0)
    base = g * BLK
    x = x_ref[...]

    rows = []
    for r_i in range(BLK):
        a = jnp.clip(aid_ref[base + r_i], 0, CAST - 1)
        rows.append(st_ref[r_i, pl.ds(a, 1), :])
    h = jnp.concatenate(rows, axis=0)

    dn = (((1,), (1,)), ((), ()))
    gi = jax.lax.dot_general(x, wiT_ref[...], dn,
                             preferred_element_type=jnp.float32) + bi_ref[...]
    gh = jax.lax.dot_general(h, whT_ref[...], dn,
                             preferred_element_type=jnp.float32) + bh_ref[...]
    i_r, i_z, i_n = gi[:, :HID], gi[:, HID:2 * HID], gi[:, 2 * HID:]
    h_r, h_z, h_n = gh[:, :HID], gh[:, HID:2 * HID], gh[:, 2 * HID:]
    r = jax.nn.sigmoid(i_r + h_r)
    z = jax.nn.sigmoid(i_z + h_z)
    n = jnp.tanh(i_n + r * h_n)
    new_h = (1.0 - z) * n + z * h
    sel_ref[...] = new_h

    out_ref[...] = st_ref[...]
    for r_i in range(BLK):
        a = jnp.clip(aid_ref[base + r_i], 0, CAST - 1)
        out_ref[r_i, pl.ds(a, 1), :] = new_h[r_i:r_i + 1, :]
    n_stop = stop_ref.shape[0]
    for r_i in range(BLK):
        rid = base + r_i
        cond = stop_ref[0] == rid
        for j in range(1, n_stop):
            cond = jnp.logical_or(cond, stop_ref[j] == rid)

        def _zero(r_i=r_i):
            out_ref[r_i] = jnp.zeros((CAST, HID), jnp.float32)
        pl.when(cond)(_zero)


def kernel(x, state, W_ih, W_hh, b_ih, b_hh, batch_idxs, actor_ids,
           story_stop_idxs):
    del batch_idxs
    aid = actor_ids.astype(jnp.int32)
    stops = story_stop_idxs.astype(jnp.int32)
    bi = b_ih.reshape(1, 3 * HID)
    bh = b_hh.reshape(1, 3 * HID)

    grid = BATCH // BLK
    new_selected, new_state = pl.pallas_call(
        _fused_body,
        grid=(grid,),
        in_specs=[
            pl.BlockSpec((BLK, INP), lambda g: (g, 0)),
            pl.BlockSpec((BLK, CAST, HID), lambda g: (g, 0, 0)),
            pl.BlockSpec((3 * HID, INP), lambda g: (0, 0)),
            pl.BlockSpec((3 * HID, HID), lambda g: (0, 0)),
            pl.BlockSpec((1, 3 * HID), lambda g: (0, 0)),
            pl.BlockSpec((1, 3 * HID), lambda g: (0, 0)),
            pl.BlockSpec(memory_space=pltpu.SMEM),
            pl.BlockSpec(memory_space=pltpu.SMEM),
        ],
        out_specs=[
            pl.BlockSpec((BLK, HID), lambda g: (g, 0)),
            pl.BlockSpec((BLK, CAST, HID), lambda g: (g, 0, 0)),
        ],
        out_shape=[
            jax.ShapeDtypeStruct((BATCH, HID), jnp.float32),
            jax.ShapeDtypeStruct((BATCH, CAST, HID), jnp.float32),
        ],
    )(x, state, W_ih, W_hh, bi, bh, aid, stops)
    return new_selected, new_state
